# Initial kernel scaffold; baseline (speedup 1.0000x reference)
import jax
import jax.numpy as jnp
from jax.experimental import pallas as pl


def kernel(input_ids, weight):
    raise NotImplementedError("write your pallas kernel here")


# speedup vs baseline: 1.6850x; 1.6850x over previous
import functools

import jax
import jax.numpy as jnp
from jax import lax
from jax.experimental import pallas as pl
from jax.experimental.pallas import tpu as pltpu
from jax.experimental.pallas import tpu_sc as plsc

BATCH = 16384
HIST = 50
HIDDEN = 64
TOTAL = BATCH * HIST

NUM_CORES = 2
NUM_SUBCORES = 16
NUM_WORKERS = NUM_CORES * NUM_SUBCORES

CHUNK = 128
ROWS_PER_WORKER = TOTAL // NUM_WORKERS
STEPS = ROWS_PER_WORKER // CHUNK


def _build_kernel():
    mesh = plsc.VectorSubcoreMesh(core_axis_name="c", subcore_axis_name="s")

    @functools.partial(
        pl.kernel,
        mesh=mesh,
        compiler_params=pltpu.CompilerParams(use_tc_tiling_on_sc=False),
        out_type=jax.ShapeDtypeStruct((TOTAL, HIDDEN), jnp.float32),
        scratch_types=[
            pltpu.VMEM((STEPS, CHUNK), jnp.int32),
            pltpu.VMEM((CHUNK, HIDDEN), jnp.float32),
            pltpu.SemaphoreType.DMA,
        ],
    )
    def emb_kernel(idx_hbm, table_hbm, out_hbm, idx_v, rows_v, sem):
        wid = lax.axis_index("s") * NUM_CORES + lax.axis_index("c")
        base_row = wid * ROWS_PER_WORKER
        pltpu.sync_copy(idx_hbm.at[pl.ds(wid * STEPS, STEPS)], idx_v)

        def body(j, carry):
            pltpu.async_copy(table_hbm.at[idx_v.at[j]], rows_v, sem).wait()
            pltpu.sync_copy(
                rows_v, out_hbm.at[pl.ds(base_row + j * CHUNK, CHUNK)]
            )
            return carry

        lax.fori_loop(0, STEPS, body, 0)

    return emb_kernel


_EMB_KERNEL = _build_kernel()


@jax.jit
def kernel(input_ids, weight):
    idx = input_ids.reshape(TOTAL // CHUNK, CHUNK).astype(jnp.int32)
    out = _EMB_KERNEL(idx, weight)
    return out.reshape(BATCH, HIST, HIDDEN)


# speedup vs baseline: 1.8763x; 1.1135x over previous
import functools

import jax
import jax.numpy as jnp
from jax import lax
from jax.experimental import pallas as pl
from jax.experimental.pallas import tpu as pltpu
from jax.experimental.pallas import tpu_sc as plsc

BATCH = 16384
HIST = 50
HIDDEN = 64
TOTAL = BATCH * HIST

NUM_CORES = 2
NUM_SUBCORES = 16
NUM_WORKERS = NUM_CORES * NUM_SUBCORES

CHUNK = 128
K = 4
ROWS_PER_BUF = CHUNK * K
ROWS_PER_WORKER = TOTAL // NUM_WORKERS
STEPS = ROWS_PER_WORKER // CHUNK
NCHUNK = STEPS // K
GROUPS = NCHUNK // 2


def _build_kernel():
    mesh = plsc.VectorSubcoreMesh(core_axis_name="c", subcore_axis_name="s")

    @functools.partial(
        pl.kernel,
        mesh=mesh,
        compiler_params=pltpu.CompilerParams(use_tc_tiling_on_sc=False),
        out_type=jax.ShapeDtypeStruct((TOTAL, HIDDEN), jnp.float32),
        scratch_types=[
            pltpu.VMEM((STEPS, CHUNK), jnp.int32),
            pltpu.VMEM((ROWS_PER_BUF, HIDDEN), jnp.float32),
            pltpu.VMEM((ROWS_PER_BUF, HIDDEN), jnp.float32),
            pltpu.SemaphoreType.DMA,
            pltpu.SemaphoreType.DMA,
        ],
    )
    def emb_kernel(idx_hbm, table_hbm, out_hbm, idx_v, buf_a, buf_b, sem_a, sem_b):
        wid = lax.axis_index("s") * NUM_CORES + lax.axis_index("c")
        base_row = wid * ROWS_PER_WORKER
        pltpu.sync_copy(idx_hbm.at[pl.ds(wid * STEPS, STEPS)], idx_v)

        def fire(c, buf, sem):
            for k in range(K):
                pltpu.async_copy(
                    table_hbm.at[idx_v.at[c * K + k]],
                    buf.at[pl.ds(k * CHUNK, CHUNK)],
                    sem,
                )

        def drain_and_write(c, buf, sem):
            for k in range(K):
                pltpu.make_async_copy(
                    table_hbm.at[pl.ds(0, CHUNK)],
                    buf.at[pl.ds(k * CHUNK, CHUNK)],
                    sem,
                ).wait()
            pltpu.sync_copy(
                buf, out_hbm.at[pl.ds(base_row + c * ROWS_PER_BUF, ROWS_PER_BUF)]
            )

        fire(0, buf_a, sem_a)

        def group(g, carry):
            for p, (buf, sem, obuf, osem) in enumerate(
                ((buf_a, sem_a, buf_b, sem_b), (buf_b, sem_b, buf_a, sem_a))
            ):
                c = 2 * g + p

                @pl.when(c + 1 < NCHUNK)
                def _():
                    fire(c + 1, obuf, osem)

                drain_and_write(c, buf, sem)
            return carry

        lax.fori_loop(0, GROUPS, group, 0)

    return emb_kernel


_EMB_KERNEL = _build_kernel()


@jax.jit
def kernel(input_ids, weight):
    idx = input_ids.reshape(TOTAL // CHUNK, CHUNK).astype(jnp.int32)
    out = _EMB_KERNEL(idx, weight)
    return out.reshape(BATCH, HIST, HIDDEN)
